# baseline (speedup 1.0000x reference)
import dataclasses
import functools

import jax
import jax.numpy as jnp
import numpy as np
from jax.experimental import pallas as pl
from jax.experimental.pallas import tpu as pltpu
from jax.experimental.pallas import tpu_sc as plsc

_BATCH = 128
_VOCAB = 100000
_TEMPERATURE = 1.0

_CH = 8
_NCH = _BATCH // _CH
_K = 4

_SC_SUBCORES = 32
_SC_ROWS = _BATCH // _SC_SUBCORES
_SC_C = 2048


def _rotl(x, r):
    return ((x << np.uint32(r)) | (x >> np.uint32(32 - r))).astype(np.uint32)


def _threefry2x32(k0, k1, x0, x1):
    x0 = x0.astype(np.uint32).copy()
    x1 = x1.astype(np.uint32).copy()
    ks0 = np.uint32(k0)
    ks1 = np.uint32(k1)
    ks2 = np.uint32(0x1BD11BDA) ^ ks0 ^ ks1
    ks = [ks0, ks1, ks2]
    rotations = [(13, 15, 26, 6), (17, 29, 16, 24)]
    x0 += ks0
    x1 += ks1
    for i in range(5):
        for r in rotations[i % 2]:
            x0 += x1
            x1 = _rotl(x1, r)
            x1 ^= x0
        x0 += ks[(i + 1) % 3]
        x1 += ks[(i + 2) % 3]
        x1 += np.uint32(i + 1)
    return x0, x1


@functools.cache
def _gumbel_table() -> np.ndarray:
    k0, k1 = _threefry2x32(
        0, 0, np.zeros(1, np.uint32), np.array([1234], np.uint32)
    )
    n = _BATCH * _VOCAB
    counts_hi = np.zeros(n, dtype=np.uint32)
    counts_lo = np.arange(n, dtype=np.uint32)
    o0, o1 = _threefry2x32(int(k0[0]), int(k1[0]), counts_hi, counts_lo)
    bits = o0 ^ o1
    float_bits = (bits >> np.uint32(9)) | np.uint32(0x3F800000)
    f = float_bits.view(np.float32) - np.float32(1.0)
    minval = np.float32(1e-20)
    u = np.maximum(minval, f * (np.float32(1.0) - minval) + minval)
    g = -np.log(-np.log(u.astype(np.float64)))
    return g.astype(np.float32).reshape(_BATCH, _VOCAB)


def _row_chunk(ref, i):
    return ref.at[pl.ds(i * _CH, _CH), :]


def _masked_kernel(logits_hbm, mask_ref, masked_hbm, lbuf, obuf, lsem, osem):
    def start_in(i, slot):
        pltpu.make_async_copy(
            _row_chunk(logits_hbm, i), lbuf.at[slot], lsem.at[slot]).start()

    for s in range(_K):
        start_in(s, s)

    mask_row = mask_ref[...]

    for i in range(_NCH):
        slot = i % _K
        pltpu.make_async_copy(
            _row_chunk(logits_hbm, i), lbuf.at[slot], lsem.at[slot]).wait()
        if i >= _K:
            pltpu.make_async_copy(
                obuf.at[slot], _row_chunk(masked_hbm, i - _K),
                osem.at[slot]).wait()
        obuf[slot] = lbuf[slot] * (1.0 / _TEMPERATURE) + mask_row
        pltpu.make_async_copy(
            obuf.at[slot], _row_chunk(masked_hbm, i), osem.at[slot]).start()
        if i + _K < _NCH:
            start_in(i + _K, slot)

    for s in range(_K):
        i = _NCH - _K + s
        pltpu.make_async_copy(
            obuf.at[s], _row_chunk(masked_hbm, i), osem.at[s]).wait()


def _masked_call(logits, mask2d):
    return pl.pallas_call(
        _masked_kernel,
        in_specs=[
            pl.BlockSpec(memory_space=pl.ANY),
            pl.BlockSpec((1, _VOCAB), lambda: (0, 0)),
        ],
        out_specs=pl.BlockSpec(memory_space=pl.ANY),
        out_shape=jax.ShapeDtypeStruct((_BATCH, _VOCAB), jnp.float32),
        scratch_shapes=[
            pltpu.VMEM((_K, _CH, _VOCAB), jnp.float32),
            pltpu.VMEM((_K, _CH, _VOCAB), jnp.float32),
            pltpu.SemaphoreType.DMA((_K,)),
            pltpu.SemaphoreType.DMA((_K,)),
        ],
    )(logits, mask2d)


def _sc_sample_call(logits3, mask2d, gumbel3):
    mesh = plsc.VectorSubcoreMesh(core_axis_name="c", subcore_axis_name="s")
    neg_inf = jnp.float32(-jnp.inf)
    nfull = _VOCAB // _SC_C
    rem = _VOCAB % _SC_C
    chunks = [(k * _SC_C, _SC_C) for k in range(nfull)]

    cp = pltpu.CompilerParams()
    if "needs_layout_passes" in pltpu.CompilerParams.__dataclass_fields__:
        cp = dataclasses.replace(cp, needs_layout_passes=False)

    @pl.kernel(
        compiler_params=cp,
        out_type=jax.ShapeDtypeStruct((_SC_SUBCORES, _SC_ROWS, 16),
                                      jnp.int32),
        mesh=mesh,
        scratch_types=[
            pltpu.VMEM((2, 8, _SC_C), jnp.float32),
            pltpu.VMEM((2, 8, _SC_C), jnp.float32),
            pltpu.VMEM((2, 1, _SC_C), jnp.float32),
            pltpu.VMEM((8, rem), jnp.float32),
            pltpu.VMEM((8, rem), jnp.float32),
            pltpu.VMEM((1, rem), jnp.float32),
            pltpu.VMEM((_SC_ROWS, 16), jnp.float32),
            pltpu.VMEM((_SC_ROWS, 16), jnp.int32),
            pltpu.VMEM((_SC_ROWS, 16), jnp.int32),
            pltpu.SemaphoreType.DMA((2,)),
            pltpu.SemaphoreType.DMA((2,)),
            pltpu.SemaphoreType.DMA((2,)),
            pltpu.SemaphoreType.DMA((3,)),
            pltpu.SemaphoreType.DMA,
        ],
    )
    def sample_kernel(l_hbm, m_hbm, g_hbm, o_hbm,
                      lbuf, gbuf, mbuf, ltail, gtail, mtail,
                      rmax, ridx, obuf,
                      lsem, gsem, msem, tsem, osem):
        sub = jax.lax.axis_index("c") * 16 + jax.lax.axis_index("s")
        grp = sub // 2
        p4 = (sub % 2) * _SC_ROWS
        lane = jax.lax.iota(jnp.int32, 16)

        def start_chunk(ci, b):
            off, sz = chunks[ci]
            pltpu.make_async_copy(
                l_hbm.at[grp, :, pl.ds(off, sz)],
                lbuf.at[b, :, pl.ds(0, sz)], lsem.at[b]).start()
            pltpu.make_async_copy(
                g_hbm.at[grp, :, pl.ds(off, sz)],
                gbuf.at[b, :, pl.ds(0, sz)], gsem.at[b]).start()
            pltpu.make_async_copy(
                m_hbm.at[:, pl.ds(off, sz)],
                mbuf.at[b, :, pl.ds(0, sz)], msem.at[b]).start()

        def wait_chunk(ci, b):
            off, sz = chunks[ci]
            pltpu.make_async_copy(
                l_hbm.at[grp, :, pl.ds(off, sz)],
                lbuf.at[b, :, pl.ds(0, sz)], lsem.at[b]).wait()
            pltpu.make_async_copy(
                g_hbm.at[grp, :, pl.ds(off, sz)],
                gbuf.at[b, :, pl.ds(0, sz)], gsem.at[b]).wait()
            pltpu.make_async_copy(
                m_hbm.at[:, pl.ds(off, sz)],
                mbuf.at[b, :, pl.ds(0, sz)], msem.at[b]).wait()

        for r in range(_SC_ROWS):
            rmax[r, :] = jnp.full((16,), neg_inf)
            ridx[r, :] = jnp.zeros((16,), jnp.int32)

        tail_off = nfull * _SC_C
        pltpu.make_async_copy(
            l_hbm.at[grp, :, pl.ds(tail_off, rem)], ltail, tsem.at[0]).start()
        pltpu.make_async_copy(
            g_hbm.at[grp, :, pl.ds(tail_off, rem)], gtail, tsem.at[1]).start()
        pltpu.make_async_copy(
            m_hbm.at[:, pl.ds(tail_off, rem)], mtail, tsem.at[2]).start()

        start_chunk(0, 0)
        for ci in range(len(chunks)):
            b = ci % 2
            if ci + 1 < len(chunks):
                start_chunk(ci + 1, (ci + 1) % 2)
            wait_chunk(ci, b)
            off, sz = chunks[ci]

            @pl.loop(0, sz // 16)
            def _(gi, b=b, off=off):
                base = off + gi * 16
                mv = mbuf[b, 0, pl.ds(gi * 16, 16)]
                for r in range(_SC_ROWS):
                    lv = lbuf[b, p4 + r, pl.ds(gi * 16, 16)]
                    gv = gbuf[b, p4 + r, pl.ds(gi * 16, 16)]
                    z = (lv * (1.0 / _TEMPERATURE) + mv) + gv
                    cur_max = rmax[r, :]
                    cur_idx = ridx[r, :]
                    better = z > cur_max
                    rmax[r, :] = jnp.where(better, z, cur_max)
                    ridx[r, :] = jnp.where(better, base + lane, cur_idx)

        pltpu.make_async_copy(
            l_hbm.at[grp, :, pl.ds(tail_off, rem)], ltail, tsem.at[0]).wait()
        pltpu.make_async_copy(
            g_hbm.at[grp, :, pl.ds(tail_off, rem)], gtail, tsem.at[1]).wait()
        pltpu.make_async_copy(
            m_hbm.at[:, pl.ds(tail_off, rem)], mtail, tsem.at[2]).wait()

        @pl.loop(0, rem // 16)
        def _(gi):
            base = tail_off + gi * 16
            mv = mtail[0, pl.ds(gi * 16, 16)]
            for r in range(_SC_ROWS):
                lv = ltail[p4 + r, pl.ds(gi * 16, 16)]
                gv = gtail[p4 + r, pl.ds(gi * 16, 16)]
                z = (lv * (1.0 / _TEMPERATURE) + mv) + gv
                cur_max = rmax[r, :]
                cur_idx = ridx[r, :]
                better = z > cur_max
                rmax[r, :] = jnp.where(better, z, cur_max)
                ridx[r, :] = jnp.where(better, base + lane, cur_idx)

        for r in range(_SC_ROWS):
            m = jnp.max(rmax[r, :])
            hit = jnp.where(rmax[r, :] == m, ridx[r, :], jnp.int32(_VOCAB))
            obuf[r, :] = jnp.full((16,), jnp.min(hit), jnp.int32)
        pltpu.make_async_copy(obuf, o_hbm.at[sub], osem).start()
        pltpu.make_async_copy(obuf, o_hbm.at[sub], osem).wait()

    return sample_kernel(logits3, mask2d, gumbel3)


def kernel(logits, prediction_mask):
    gumbel = jnp.asarray(_gumbel_table())
    mask2d = prediction_mask.reshape(1, _VOCAB)
    masked = _masked_call(logits, mask2d)
    ids32 = _sc_sample_call(
        logits.reshape(16, 8, _VOCAB), mask2d,
        gumbel.reshape(16, 8, _VOCAB))
    return ids32[:, :, 0].reshape(_BATCH), masked


# speedup vs baseline: 2.2522x; 2.2522x over previous
import functools

import jax
import jax.numpy as jnp
import numpy as np
from jax.experimental import pallas as pl
from jax.experimental.pallas import tpu as pltpu

_BATCH = 128
_VOCAB = 100000
_TEMPERATURE = 1.0
_CH = 8
_NCH = _BATCH // _CH
_K = 4


def _rotl(x, r):
    return ((x << np.uint32(r)) | (x >> np.uint32(32 - r))).astype(np.uint32)


def _threefry2x32(k0, k1, x0, x1):
    x0 = x0.astype(np.uint32).copy()
    x1 = x1.astype(np.uint32).copy()
    ks0 = np.uint32(k0)
    ks1 = np.uint32(k1)
    ks2 = np.uint32(0x1BD11BDA) ^ ks0 ^ ks1
    ks = [ks0, ks1, ks2]
    rotations = [(13, 15, 26, 6), (17, 29, 16, 24)]
    x0 += ks0
    x1 += ks1
    for i in range(5):
        for r in rotations[i % 2]:
            x0 += x1
            x1 = _rotl(x1, r)
            x1 ^= x0
        x0 += ks[(i + 1) % 3]
        x1 += ks[(i + 2) % 3]
        x1 += np.uint32(i + 1)
    return x0, x1


@functools.cache
def _gumbel_table() -> np.ndarray:
    k0, k1 = _threefry2x32(
        0, 0, np.zeros(1, np.uint32), np.array([1234], np.uint32)
    )
    n = _BATCH * _VOCAB
    counts_hi = np.zeros(n, dtype=np.uint32)
    counts_lo = np.arange(n, dtype=np.uint32)
    o0, o1 = _threefry2x32(int(k0[0]), int(k1[0]), counts_hi, counts_lo)
    bits = o0 ^ o1
    float_bits = (bits >> np.uint32(9)) | np.uint32(0x3F800000)
    f = float_bits.view(np.float32) - np.float32(1.0)
    minval = np.float32(1e-20)
    u = np.maximum(minval, f * (np.float32(1.0) - minval) + minval)
    g = -np.log(-np.log(u.astype(np.float64)))
    return g.astype(np.float32).reshape(_BATCH, _VOCAB)


def _row_chunk(ref, i):
    return ref.at[pl.ds(i * _CH, _CH), :]


def _sample_kernel(logits_hbm, mask_ref, gumbel_hbm, masked_hbm, ids_ref,
                   lbuf, gbuf, obuf, lsem, gsem, osem):
    def start_in(i, slot):
        pltpu.make_async_copy(
            _row_chunk(logits_hbm, i), lbuf.at[slot], lsem.at[slot]).start()
        pltpu.make_async_copy(
            _row_chunk(gumbel_hbm, i), gbuf.at[slot], gsem.at[slot]).start()

    for s in range(_K):
        start_in(s, s)

    mask_row = mask_ref[...]

    for i in range(_NCH):
        slot = i % _K
        pltpu.make_async_copy(
            _row_chunk(logits_hbm, i), lbuf.at[slot], lsem.at[slot]).wait()
        pltpu.make_async_copy(
            _row_chunk(gumbel_hbm, i), gbuf.at[slot], gsem.at[slot]).wait()
        if i >= _K:
            pltpu.make_async_copy(
                obuf.at[slot], _row_chunk(masked_hbm, i - _K),
                osem.at[slot]).wait()
        masked = lbuf[slot] * (1.0 / _TEMPERATURE) + mask_row
        obuf[slot] = masked
        z = masked + gbuf[slot]
        best = jnp.max(z, axis=1, keepdims=True)
        idx = jax.lax.broadcasted_iota(jnp.int32, z.shape, 1)
        hit = jnp.where(z == best, idx, jnp.int32(_VOCAB))
        ids_ref[pl.ds(i * _CH, _CH), :] = jnp.min(hit, axis=1, keepdims=True)
        pltpu.make_async_copy(
            obuf.at[slot], _row_chunk(masked_hbm, i), osem.at[slot]).start()
        if i + _K < _NCH:
            start_in(i + _K, slot)

    for s in range(_K):
        i = _NCH - _K + s
        pltpu.make_async_copy(
            obuf.at[s], _row_chunk(masked_hbm, i), osem.at[s]).wait()


def kernel(logits, prediction_mask):
    gumbel = jnp.asarray(_gumbel_table())
    mask2d = prediction_mask.reshape(1, _VOCAB)
    masked, ids = pl.pallas_call(
        _sample_kernel,
        in_specs=[
            pl.BlockSpec(memory_space=pl.ANY),
            pl.BlockSpec((1, _VOCAB), lambda: (0, 0)),
            pl.BlockSpec(memory_space=pl.ANY),
        ],
        out_specs=[
            pl.BlockSpec(memory_space=pl.ANY),
            pl.BlockSpec((_BATCH, 1), lambda: (0, 0)),
        ],
        out_shape=[
            jax.ShapeDtypeStruct((_BATCH, _VOCAB), jnp.float32),
            jax.ShapeDtypeStruct((_BATCH, 1), jnp.int32),
        ],
        scratch_shapes=[
            pltpu.VMEM((_K, _CH, _VOCAB), jnp.float32),
            pltpu.VMEM((_K, _CH, _VOCAB), jnp.float32),
            pltpu.VMEM((_K, _CH, _VOCAB), jnp.float32),
            pltpu.SemaphoreType.DMA((_K,)),
            pltpu.SemaphoreType.DMA((_K,)),
            pltpu.SemaphoreType.DMA((_K,)),
        ],
    )(logits, mask2d, gumbel)
    return ids.reshape(_BATCH), masked
